# Initial kernel scaffold; baseline (speedup 1.0000x reference)
import jax
import jax.numpy as jnp
from jax.experimental import pallas as pl


def kernel(x, dw_w, bn_gamma, bn_beta, bn_mean, bn_var, pw_w, pw_b, guidance_scale):
    raise NotImplementedError("write your pallas kernel here")


# speedup vs baseline: 11.1081x; 11.1081x over previous
import functools

import jax
import jax.numpy as jnp
from jax import lax
from jax.experimental import pallas as pl
from jax.experimental.pallas import tpu as pltpu

_REGION = 4


def _grid_dims(height, width, region_size):
    gh = max(height // region_size, 1)
    gw = max(width // region_size, 1)
    gh = min(gh, height)
    gw = min(gw, width)
    while gh > 1 and height % gh != 0:
        gh -= 1
    while gw > 1 and width % gw != 0:
        gw -= 1
    return gh, gw


def _rank_body(chunk, srow_ref, rank_ref):
    g = srow_ref.shape[2]
    srow = srow_ref[0]
    jrow = lax.broadcasted_iota(jnp.int32, (1, g), 1)
    for ic in range(g // chunk):
        si = srow[:, ic * chunk:(ic + 1) * chunk].T
        icol = lax.broadcasted_iota(jnp.int32, (chunk, 1), 0) + ic * chunk
        beats = (srow > si) | ((srow == si) & (jrow < icol))
        cnt = jnp.sum(beats.astype(jnp.int32), axis=1, keepdims=True)
        rank_ref[0, :, ic * chunk:(ic + 1) * chunk] = cnt.T


def _compute_rank(scores, g, chunk):
    b = scores.shape[0]
    return pl.pallas_call(
        functools.partial(_rank_body, chunk),
        grid=(b,),
        in_specs=[
            pl.BlockSpec((1, 1, g), lambda i: (i, 0, 0)),
        ],
        out_specs=pl.BlockSpec((1, 1, g), lambda i: (i, 0, 0)),
        out_shape=jax.ShapeDtypeStruct((b, 1, g), jnp.int32),
    )(scores.reshape(b, 1, g))


def _permute_body(nsteps, gw, rs, rank_ref, x_ref, out_ref, m_ref, sem_ref):
    c = x_ref.shape[1]
    w = x_ref.shape[4]
    b = pl.program_id(0)
    gh = pl.program_id(1)
    step = b * pl.num_programs(1) + gh
    p = step % 2

    def issue(t, slot, do_wait):
        tb = t // pl.num_programs(1)
        for j in range(gw):
            dst = rank_ref[t * gw + j]
            cp = pltpu.make_async_copy(
                m_ref.at[slot, j],
                out_ref.at[tb, dst],
                sem_ref.at[slot],
            )
            if do_wait:
                cp.wait()
            else:
                cp.start()

    @pl.when(step >= 2)
    def _():
        issue(step - 2, p, do_wait=True)

    m = x_ref[0, :, 0]
    t = m.reshape(c * rs, w).T
    m_ref[p] = t.reshape(gw, rs, c * rs)

    issue(step, p, do_wait=False)

    @pl.when(step == nsteps - 1)
    def _():
        issue(step - 1, (step - 1) % 2, do_wait=True)
        issue(step, p, do_wait=True)


def _permute(rank_flat, x5, b, c, gh, gw, rs, w):
    g = gh * gw
    grid_spec = pltpu.PrefetchScalarGridSpec(
        num_scalar_prefetch=1,
        grid=(b, gh),
        in_specs=[
            pl.BlockSpec((1, c, 1, rs, w), lambda bi, gi, rank_ref: (bi, 0, gi, 0, 0)),
        ],
        out_specs=pl.BlockSpec(memory_space=pl.ANY),
        scratch_shapes=[
            pltpu.VMEM((2, gw, rs, c * rs), jnp.float32),
            pltpu.SemaphoreType.DMA((2,)),
        ],
    )
    return pl.pallas_call(
        functools.partial(_permute_body, b * gh, gw, rs),
        grid_spec=grid_spec,
        out_shape=jax.ShapeDtypeStruct((b, g, rs, c * rs), jnp.float32),
    )(rank_flat, x5)


def _unpermute_body(gw, rs, t_ref, out_ref):
    c = out_ref.shape[1]
    w = out_ref.shape[4]
    t = t_ref[0, 0]
    out_ref[0, :, 0] = t.T.reshape(c, rs, w)


def _unpermute(table, b, c, gh, gw, rs, h, w):
    return pl.pallas_call(
        functools.partial(_unpermute_body, gw, rs),
        grid=(b, gh),
        in_specs=[
            pl.BlockSpec((1, 1, gw * rs, c * rs), lambda bi, gi: (bi, gi, 0, 0)),
        ],
        out_specs=pl.BlockSpec((1, c, 1, rs, w), lambda bi, gi: (bi, 0, gi, 0, 0)),
        out_shape=jax.ShapeDtypeStruct((b, c, gh, rs, w), jnp.float32),
    )(table.reshape(b, gh, gw * rs, c * rs)).reshape(b, c, h, w)


def kernel(x, dw_w, bn_gamma, bn_beta, bn_mean, bn_var, pw_w, pw_b, guidance_scale):
    b, c, h, w = x.shape
    rs = _REGION
    gh, gw = _grid_dims(h, w, rs)
    rh, rw = h // gh, w // gw
    g = gh * gw

    y = lax.conv_general_dilated(x, dw_w, (1, 1), ((1, 1), (1, 1)),
                                 dimension_numbers=("NCHW", "OIHW", "NCHW"),
                                 feature_group_count=c)
    y = (y - bn_mean[None, :, None, None]) / jnp.sqrt(bn_var[None, :, None, None] + 1e-5)
    y = y * bn_gamma[None, :, None, None] + bn_beta[None, :, None, None]
    y = y * jax.nn.sigmoid(y)
    y = lax.conv_general_dilated(y, pw_w, (1, 1), ((0, 0), (0, 0)),
                                 dimension_numbers=("NCHW", "OIHW", "NCHW"))
    y = y + pw_b[None, :, None, None]
    importance = jax.nn.sigmoid(y)
    scores = importance.reshape(b, 1, gh, rh, gw, rw).mean(axis=(3, 5)).reshape(b, g)
    scores = lax.optimization_barrier(scores)
    xm = x * (1.0 + guidance_scale * importance)

    rank = _compute_rank(scores, g, chunk=112)
    rank_flat = rank.reshape(b * g)

    xm5 = xm.reshape(b, c, gh, rh, w)
    table = _permute(rank_flat, xm5, b, c, gh, gw, rs, w)
    return _unpermute(table, b, c, gh, gw, rs, h, w)
